# baseline (device time: 196255 ns/iter reference)
import jax
import jax.numpy as jnp
from jax import lax
from jax.experimental import pallas as pl
from jax.experimental.pallas import tpu as pltpu

N_DEV = 16
BLK = 512


def kernel(x):
    m, n_total = x.shape
    assert m == BLK and n_total == N_DEV * BLK, (m, n_total)

    def body(x_ref, out_ref, send_sems, recv_sems):
        my = lax.axis_index("i")

        half = BLK // 2
        for h in range(2):
            for d in sorted(range(1, N_DEV), key=lambda d: -min(d, N_DEV - d)):
                j = (my + d) % N_DEV
                rdma = pltpu.make_async_remote_copy(
                    src_ref=x_ref.at[pl.ds(h * half, half), pl.ds(j * BLK, BLK)],
                    dst_ref=out_ref.at[pl.ds(my * BLK + h * half, half), :],
                    send_sem=send_sems.at[j, h],
                    recv_sem=recv_sems.at[my, h],
                    device_id=(j,),
                    device_id_type=pl.DeviceIdType.MESH,
                )
                rdma.start()

        for j in range(N_DEV):

            @pl.when(j == my)
            def _():
                out_ref[j * BLK : (j + 1) * BLK, :] = x_ref[
                    :, j * BLK : (j + 1) * BLK
                ]

        for h in range(2):
            for s in range(N_DEV):

                @pl.when(s != my)
                def _():
                    rdma = pltpu.make_async_remote_copy(
                        src_ref=x_ref.at[pl.ds(0, half), pl.ds(s * BLK, BLK)],
                        dst_ref=out_ref.at[pl.ds(s * BLK + h * half, half), :],
                        send_sem=send_sems.at[s, h],
                        recv_sem=recv_sems.at[s, h],
                        device_id=(s,),
                        device_id_type=pl.DeviceIdType.MESH,
                    )
                    rdma.wait_recv()

        for h in range(2):
            for j in range(N_DEV):

                @pl.when(j != my)
                def _():
                    rdma = pltpu.make_async_remote_copy(
                        src_ref=x_ref.at[pl.ds(h * half, half), pl.ds(j * BLK, BLK)],
                        dst_ref=out_ref.at[pl.ds(my * BLK + h * half, half), :],
                        send_sem=send_sems.at[j, h],
                        recv_sem=recv_sems.at[my, h],
                        device_id=(j,),
                        device_id_type=pl.DeviceIdType.MESH,
                    )
                    rdma.wait_send()

    return pl.pallas_call(
        body,
        out_shape=jax.ShapeDtypeStruct((N_DEV * BLK, BLK), x.dtype),
        in_specs=[pl.BlockSpec(memory_space=pltpu.VMEM)],
        out_specs=pl.BlockSpec(memory_space=pltpu.VMEM),
        scratch_shapes=[
            pltpu.SemaphoreType.DMA((N_DEV, 2)),
            pltpu.SemaphoreType.DMA((N_DEV, 2)),
        ],
    )(x)


# device time: 184251 ns/iter; 1.0652x vs baseline; 1.0652x over previous
import jax
import jax.numpy as jnp
from jax import lax
from jax.experimental import pallas as pl
from jax.experimental.pallas import tpu as pltpu

N_DEV = 16
BLK = 512


def kernel(x):
    m, n_total = x.shape
    assert m == BLK and n_total == N_DEV * BLK, (m, n_total)

    def body(x_ref, out_ref, send_sems, recv_sems):
        my = lax.axis_index("i")

        for d in sorted(range(1, N_DEV), key=lambda d: -min(d, N_DEV - d)):
            j = (my + d) % N_DEV
            rdma = pltpu.make_async_remote_copy(
                src_ref=x_ref.at[:, pl.ds(j * BLK, BLK)],
                dst_ref=out_ref.at[pl.ds(my * BLK, BLK), :],
                send_sem=send_sems.at[j],
                recv_sem=recv_sems.at[my],
                device_id=(j,),
                device_id_type=pl.DeviceIdType.MESH,
            )
            rdma.start()

        for j in range(N_DEV):

            @pl.when(j == my)
            def _():
                out_ref[j * BLK : (j + 1) * BLK, :] = x_ref[
                    :, j * BLK : (j + 1) * BLK
                ]

        for s in range(N_DEV):

            @pl.when(s != my)
            def _():
                rdma = pltpu.make_async_remote_copy(
                    src_ref=x_ref.at[:, pl.ds(s * BLK, BLK)],
                    dst_ref=out_ref.at[pl.ds(s * BLK, BLK), :],
                    send_sem=send_sems.at[s],
                    recv_sem=recv_sems.at[s],
                    device_id=(s,),
                    device_id_type=pl.DeviceIdType.MESH,
                )
                rdma.wait_recv()

        for j in range(N_DEV):

            @pl.when(j != my)
            def _():
                rdma = pltpu.make_async_remote_copy(
                    src_ref=x_ref.at[:, pl.ds(j * BLK, BLK)],
                    dst_ref=out_ref.at[pl.ds(my * BLK, BLK), :],
                    send_sem=send_sems.at[j],
                    recv_sem=recv_sems.at[my],
                    device_id=(j,),
                    device_id_type=pl.DeviceIdType.MESH,
                )
                rdma.wait_send()

    return pl.pallas_call(
        body,
        out_shape=jax.ShapeDtypeStruct((N_DEV * BLK, BLK), x.dtype),
        in_specs=[pl.BlockSpec(memory_space=pltpu.VMEM)],
        out_specs=pl.BlockSpec(memory_space=pltpu.VMEM),
        scratch_shapes=[
            pltpu.SemaphoreType.DMA((N_DEV,)),
            pltpu.SemaphoreType.DMA((N_DEV,)),
        ],
    )(x)
